# baseline (speedup 1.0000x reference)
import functools

import jax
import jax.numpy as jnp
import numpy as np
from jax import lax
from jax.experimental import pallas as pl
from jax.experimental.pallas import tpu as pltpu
from jax.experimental.pallas import tpu_sc as plsc

N = 10000
D = 128
E = 320000

NUM_CORES = 2
NUM_SUBCORES = 16
NUM_TILES = NUM_CORES * NUM_SUBCORES
EDGES_PER_TILE = E // NUM_TILES
CHUNK = 80
NCH = EDGES_PER_TILE // CHUNK
RG = 3
RM = 2
RE = 6
LOOK = 2
UNROLL = 6
NPAD = 10240
ROWS_PER_TILE = NPAD // NUM_SUBCORES
LANES = 16

_PERM = np.empty(D, dtype=np.int32)
for _d in range(D // 32):
    for _j in range(16):
        _PERM[32 * _d + _j] = 32 * _d + 2 * _j
        _PERM[32 * _d + 16 + _j] = 32 * _d + 2 * _j + 1


def _sc_aggregate(x16, src, dst3, ew):
    mesh = plsc.VectorSubcoreMesh(core_axis_name="c", subcore_axis_name="s")

    @functools.partial(
        pl.kernel,
        out_type=jax.ShapeDtypeStruct((NUM_CORES, NPAD, D), jnp.float32),
        mesh=mesh,
        compiler_params=pltpu.CompilerParams(use_tc_tiling_on_sc=False),
        scratch_types=[
            pltpu.VMEM((RG, CHUNK), jnp.int32),
            pltpu.VMEM((NCH, CHUNK), jnp.int32),
            pltpu.VMEM((RE, CHUNK), jnp.float32),
            pltpu.VMEM((RG, CHUNK, D // 2), jnp.int32),
            pltpu.VMEM((RM, CHUNK, D), jnp.float32),
            pltpu.VMEM_SHARED((NPAD, D), jnp.float32),
            pltpu.SemaphoreType.DMA((RG,)),
            pltpu.SemaphoreType.DMA((RE,)),
            pltpu.SemaphoreType.DMA((RG,)),
            pltpu.SemaphoreType.DMA((RM,)),
        ],
    )
    def k(x_hbm, src_hbm, dst_hbm, ew_hbm, out_hbm, src_v, dst_v, ew_v, rows_v,
          msgs_v, acc_sh, semsrc, semew, semg, sems):
        c = lax.axis_index("c")
        s = lax.axis_index("s")
        wid = s * NUM_CORES + c
        e0 = wid * EDGES_PER_TILE

        def zrow(i, carry):
            for d in range(D // LANES):
                msgs_v[0, i, pl.ds(d * LANES, LANES)] = jnp.zeros((LANES,), jnp.float32)
            return carry

        lax.fori_loop(0, CHUNK, zrow, 0)
        row0 = s * ROWS_PER_TILE
        for r in range(ROWS_PER_TILE // CHUNK):
            pltpu.sync_copy(msgs_v.at[0], acc_sh.at[pl.ds(row0 + r * CHUNK, CHUNK)])

        pltpu.sync_copy(dst_hbm.at[wid], dst_v)

        def start_src(j, p):
            pltpu.async_copy(
                src_hbm.at[pl.ds(e0 + j * CHUNK, CHUNK)], src_v.at[p], semsrc.at[p]
            )

        def wait_src(j, p):
            pltpu.make_async_copy(
                src_hbm.at[pl.ds(e0 + j * CHUNK, CHUNK)], src_v.at[p], semsrc.at[p]
            ).wait()

        def start_ew(j, p):
            pltpu.async_copy(
                ew_hbm.at[pl.ds(e0 + j * CHUNK, CHUNK)], ew_v.at[p], semew.at[p]
            )

        def wait_ew(j, p):
            pltpu.make_async_copy(
                ew_hbm.at[pl.ds(e0 + j * CHUNK, CHUNK)], ew_v.at[p], semew.at[p]
            ).wait()

        def start_gather(j, p):
            pltpu.async_copy(x_hbm.at[src_v.at[p]], rows_v.at[p], semg.at[p])

        def wait_gather(j, p):
            pltpu.make_async_copy(
                x_hbm.at[src_v.at[p]], rows_v.at[p], semg.at[p]
            ).wait()

        def start_scatter(j, pm):
            pltpu.async_copy(
                msgs_v.at[pm], acc_sh.at[dst_v.at[j]], sems.at[pm], add=True
            )

        def wait_scatter(j, pm):
            pltpu.make_async_copy(
                msgs_v.at[pm], acc_sh.at[dst_v.at[j]], sems.at[pm]
            ).wait()

        plsc.subcore_barrier()

        for k0 in range(LOOK + 1):
            start_src(k0, k0 % RG)
            start_ew(k0, k0 % RE)
        for k0 in range(LOOK):
            wait_src(k0, k0 % RG)
            start_gather(k0, k0 % RG)

        def chunk_body(j, pg, pm, pe):
            nj = j + LOOK

            @pl.when(nj < NCH)
            def _():
                wait_src(nj, (pg + LOOK) % RG)
                start_gather(nj, (pg + LOOK) % RG)

            wait_gather(j, pg)
            wait_ew(j, pe)

            mj = j + LOOK + 1

            @pl.when(mj < NCH)
            def _():
                start_src(mj, (pg + LOOK + 1) % RG)
                start_ew(mj, (pe + LOOK + 1) % RE)

            @pl.when(j >= RM)
            def _():
                wait_scatter(j - RM, pm)

            def scale_group(g, carry):
                wv = ew_v[pe, pl.ds(g * LANES, LANES)]
                mask = jnp.full((LANES,), jnp.int32(-65536))
                for e in range(LANES):
                    w = jnp.broadcast_to(wv[e], (LANES,))
                    row = g * LANES + e
                    for d in range(D // 32):
                        vi = rows_v[pg, row, pl.ds(d * LANES, LANES)]
                        even = lax.bitcast_convert_type(vi << 16, jnp.float32)
                        odd = lax.bitcast_convert_type(vi & mask, jnp.float32)
                        msgs_v[pm, row, pl.ds(d * 32, LANES)] = even * w
                        msgs_v[pm, row, pl.ds(d * 32 + LANES, LANES)] = odd * w
                return carry

            lax.fori_loop(0, CHUNK // LANES, scale_group, 0)
            start_scatter(j, pm)

        def ring_step(t, carry):
            for i in range(UNROLL):
                j = t * UNROLL + i
                chunk_body(j, i % RG, i % RM, i % RE)
            return carry

        full = (NCH // UNROLL) * UNROLL
        lax.fori_loop(0, NCH // UNROLL, ring_step, 0)
        for j in range(full, NCH):
            chunk_body(jnp.int32(j), j % RG, j % RM, j % RE)

        for j in range(NCH - RM, NCH):
            wait_scatter(jnp.int32(j), j % RM)

        plsc.subcore_barrier()

        pltpu.sync_copy(
            acc_sh.at[pl.ds(row0, ROWS_PER_TILE)],
            out_hbm.at[c, pl.ds(row0, ROWS_PER_TILE)],
        )

    return k(x16, src, dst3, ew)


def _tc_finish(parts, Wp, b2):
    blk = 1000

    def body(p_ref, w_ref, b_ref, o_ref):
        acc = p_ref[0] + p_ref[1]
        h = jnp.dot(acc, w_ref[...], preferred_element_type=jnp.float32)
        o_ref[...] = jnp.maximum(h + b_ref[...], 0.0)

    return pl.pallas_call(
        body,
        grid=(N // blk,),
        in_specs=[
            pl.BlockSpec((NUM_CORES, blk, D), lambda i: (0, i, 0)),
            pl.BlockSpec((D, D), lambda i: (0, 0)),
            pl.BlockSpec((1, D), lambda i: (0, 0)),
        ],
        out_specs=pl.BlockSpec((blk, D), lambda i: (i, 0)),
        out_shape=jax.ShapeDtypeStruct((N, D), jnp.float32),
    )(parts, Wp, b2)


def kernel(x, edge_index, edge_weight, W, b):
    ei = edge_index.astype(jnp.int32)
    src = ei[0]
    dst3 = ei[1].reshape(NUM_TILES, NCH, CHUNK)
    x16 = x.astype(jnp.bfloat16)
    xp = jax.lax.bitcast_convert_type(x16.reshape(N, D // 2, 2), jnp.int32)
    parts = _sc_aggregate(xp, src, dst3, edge_weight)
    Wp = W[jnp.asarray(_PERM), :]
    return _tc_finish(parts, Wp, b.reshape(1, D))


# speedup vs baseline: 1.9614x; 1.9614x over previous
import functools

import jax
import jax.numpy as jnp
from jax import lax
from jax.experimental import pallas as pl
from jax.experimental.pallas import tpu as pltpu
from jax.experimental.pallas import tpu_sc as plsc

N = 10000
D = 128
E = 320000

NUM_CORES = 2
NUM_SUBCORES = 16
NUM_TILES = NUM_CORES * NUM_SUBCORES
EDGES_PER_TILE = E // NUM_TILES
CHUNK = 80
NCH = EDGES_PER_TILE // CHUNK
RING = 3
NPAD = 10240
ROWS_PER_TILE = NPAD // NUM_SUBCORES
LANES = 16
D_BLKS = D // LANES


def _sc_aggregate(x, src, dst, ew):
    mesh = plsc.VectorSubcoreMesh(core_axis_name="c", subcore_axis_name="s")

    @functools.partial(
        pl.kernel,
        out_type=jax.ShapeDtypeStruct((NUM_CORES, NPAD, D), jnp.float32),
        mesh=mesh,
        scratch_types=[
            pltpu.VMEM((RING, CHUNK), jnp.int32),
            pltpu.VMEM((RING, CHUNK), jnp.int32),
            pltpu.VMEM((RING, CHUNK), jnp.float32),
            pltpu.VMEM((RING, CHUNK, D), jnp.float32),
            pltpu.VMEM_SHARED((NPAD, D), jnp.float32),
            pltpu.SemaphoreType.DMA((RING,)),
            pltpu.SemaphoreType.DMA((RING,)),
            pltpu.SemaphoreType.DMA((RING,)),
            pltpu.SemaphoreType.DMA((RING,)),
            pltpu.SemaphoreType.DMA((RING,)),
        ],
    )
    def k(x_hbm, src_hbm, dst_hbm, ew_hbm, out_hbm, src_v, dst_v, ew_v, rows_v,
          acc_sh, semsrc, semdst, semew, semg, sems):
        c = lax.axis_index("c")
        s = lax.axis_index("s")
        wid = s * NUM_CORES + c
        e0 = wid * EDGES_PER_TILE

        def zrow(i, carry):
            for d in range(D_BLKS):
                rows_v[0, i, pl.ds(d * LANES, LANES)] = jnp.zeros((LANES,), jnp.float32)
            return carry

        lax.fori_loop(0, CHUNK, zrow, 0)
        row0 = s * ROWS_PER_TILE
        for r in range(ROWS_PER_TILE // CHUNK):
            pltpu.sync_copy(rows_v.at[0], acc_sh.at[pl.ds(row0 + r * CHUNK, CHUNK)])

        def start_idx(j, p, hbm, ring, sem):
            pltpu.async_copy(hbm.at[pl.ds(e0 + j * CHUNK, CHUNK)], ring.at[p], sem.at[p])

        def wait_idx(j, p, hbm, ring, sem):
            pltpu.make_async_copy(
                hbm.at[pl.ds(e0 + j * CHUNK, CHUNK)], ring.at[p], sem.at[p]
            ).wait()

        def start_gather(j, p):
            pltpu.async_copy(x_hbm.at[src_v.at[p]], rows_v.at[p], semg.at[p])

        def wait_gather(j, p):
            pltpu.make_async_copy(x_hbm.at[src_v.at[p]], rows_v.at[p], semg.at[p]).wait()

        def start_scatter(j, p):
            pltpu.async_copy(rows_v.at[p], acc_sh.at[dst_v.at[p]], sems.at[p], add=True)

        def wait_scatter(j, p):
            pltpu.make_async_copy(rows_v.at[p], acc_sh.at[dst_v.at[p]], sems.at[p]).wait()

        plsc.subcore_barrier()

        for k0 in range(2):
            start_idx(k0, k0, src_hbm, src_v, semsrc)
            start_idx(k0, k0, dst_hbm, dst_v, semdst)
            start_idx(k0, k0, ew_hbm, ew_v, semew)
        for k0 in range(2):
            wait_idx(k0, k0, src_hbm, src_v, semsrc)
            start_gather(k0, k0)

        def chunk_body(j, p):
            nj = j + 2
            np_ = (p + 2) % RING

            wait_gather(j, p)

            @pl.when(nj < NCH)
            def _():
                start_idx(nj, np_, src_hbm, src_v, semsrc)
                start_idx(nj, np_, ew_hbm, ew_v, semew)

            wait_idx(j, p, ew_hbm, ew_v, semew)

            def scale_group(g, carry):
                wv = ew_v[p, pl.ds(g * LANES, LANES)]
                for e in range(LANES):
                    w = jnp.broadcast_to(wv[e], (LANES,))
                    row = g * LANES + e
                    for d in range(D_BLKS):
                        rows_v[p, row, pl.ds(d * LANES, LANES)] = (
                            rows_v[p, row, pl.ds(d * LANES, LANES)] * w
                        )
                return carry

            lax.fori_loop(0, CHUNK // LANES, scale_group, 0)
            wait_idx(j, p, dst_hbm, dst_v, semdst)
            start_scatter(j, p)

            @pl.when(j >= 1)
            def _():
                wait_scatter(j - 1, np_)

            @pl.when(nj < NCH)
            def _():
                start_idx(nj, np_, dst_hbm, dst_v, semdst)
                wait_idx(nj, np_, src_hbm, src_v, semsrc)
                start_gather(nj, np_)

        def ring_step(t, carry):
            for i in range(RING):
                chunk_body(t * RING + i, i)
            return carry

        full = (NCH // RING) * RING
        lax.fori_loop(0, NCH // RING, ring_step, 0)
        for j in range(full, NCH):
            chunk_body(jnp.int32(j), j % RING)

        wait_scatter(jnp.int32(NCH - 1), (NCH - 1) % RING)

        plsc.subcore_barrier()

        pltpu.sync_copy(
            acc_sh.at[pl.ds(row0, ROWS_PER_TILE)],
            out_hbm.at[c, pl.ds(row0, ROWS_PER_TILE)],
        )

    return k(x, src, dst, ew)


def _tc_finish(parts, W, b2):
    blk = 1000

    def body(p_ref, w_ref, b_ref, o_ref):
        acc = p_ref[0] + p_ref[1]
        h = jnp.dot(acc, w_ref[...], preferred_element_type=jnp.float32)
        o_ref[...] = jnp.maximum(h + b_ref[...], 0.0)

    return pl.pallas_call(
        body,
        grid=(N // blk,),
        in_specs=[
            pl.BlockSpec((NUM_CORES, blk, D), lambda i: (0, i, 0)),
            pl.BlockSpec((D, D), lambda i: (0, 0)),
            pl.BlockSpec((1, D), lambda i: (0, 0)),
        ],
        out_specs=pl.BlockSpec((blk, D), lambda i: (i, 0)),
        out_shape=jax.ShapeDtypeStruct((N, D), jnp.float32),
    )(parts, W, b2)


def kernel(x, edge_index, edge_weight, W, b):
    ei = edge_index.astype(jnp.int32)
    parts = _sc_aggregate(x, ei[0], ei[1], edge_weight)
    return _tc_finish(parts, W, b.reshape(1, D))


# speedup vs baseline: 2.0538x; 1.0471x over previous
import functools

import jax
import jax.numpy as jnp
from jax import lax
from jax.experimental import pallas as pl
from jax.experimental.pallas import tpu as pltpu
from jax.experimental.pallas import tpu_sc as plsc

N = 10000
D = 128
E = 320000

NUM_CORES = 2
NUM_SUBCORES = 16
NUM_TILES = NUM_CORES * NUM_SUBCORES
EDGES_PER_TILE = E // NUM_TILES
CHUNK = 80
NCH = EDGES_PER_TILE // CHUNK
RING = 3
NPAD = 10240
ROWS_PER_TILE = NPAD // NUM_SUBCORES
LANES = 16
D_BLKS = D // LANES


def _sc_aggregate(x, src, dst, ew):
    mesh = plsc.VectorSubcoreMesh(core_axis_name="c", subcore_axis_name="s")

    @functools.partial(
        pl.kernel,
        out_type=jax.ShapeDtypeStruct((NUM_CORES, NPAD, D), jnp.float32),
        mesh=mesh,
        scratch_types=[
            pltpu.VMEM((EDGES_PER_TILE,), jnp.int32),
            pltpu.VMEM((RING, CHUNK), jnp.int32),
            pltpu.VMEM((RING, CHUNK), jnp.float32),
            pltpu.VMEM((RING, CHUNK, D), jnp.float32),
            pltpu.VMEM_SHARED((NPAD, D), jnp.float32),
            pltpu.SemaphoreType.DMA((RING,)),
            pltpu.SemaphoreType.DMA((RING,)),
            pltpu.SemaphoreType.DMA((RING,)),
            pltpu.SemaphoreType.DMA((RING,)),
        ],
    )
    def k(x_hbm, src_hbm, dst_hbm, ew_hbm, out_hbm, src_v, dst_v, ew_v, rows_v,
          acc_sh, semdst, semew, semg, sems):
        c = lax.axis_index("c")
        s = lax.axis_index("s")
        wid = s * NUM_CORES + c
        e0 = wid * EDGES_PER_TILE

        def zrow(i, carry):
            for d in range(D_BLKS):
                rows_v[0, i, pl.ds(d * LANES, LANES)] = jnp.zeros((LANES,), jnp.float32)
            return carry

        lax.fori_loop(0, CHUNK, zrow, 0)
        row0 = s * ROWS_PER_TILE
        for r in range(ROWS_PER_TILE // CHUNK):
            pltpu.sync_copy(rows_v.at[0], acc_sh.at[pl.ds(row0 + r * CHUNK, CHUNK)])

        pltpu.sync_copy(src_hbm.at[pl.ds(e0, EDGES_PER_TILE)], src_v)

        def start_idx(j, p, hbm, ring, sem):
            pltpu.async_copy(hbm.at[pl.ds(e0 + j * CHUNK, CHUNK)], ring.at[p], sem.at[p])

        def wait_idx(j, p, hbm, ring, sem):
            pltpu.make_async_copy(
                hbm.at[pl.ds(e0 + j * CHUNK, CHUNK)], ring.at[p], sem.at[p]
            ).wait()

        def start_gather(j, p):
            pltpu.async_copy(
                x_hbm.at[src_v.at[pl.ds(j * CHUNK, CHUNK)]], rows_v.at[p], semg.at[p]
            )

        def wait_gather(j, p):
            pltpu.make_async_copy(
                x_hbm.at[src_v.at[pl.ds(j * CHUNK, CHUNK)]], rows_v.at[p], semg.at[p]
            ).wait()

        def start_scatter(j, p):
            pltpu.async_copy(rows_v.at[p], acc_sh.at[dst_v.at[p]], sems.at[p], add=True)

        def wait_scatter(j, p):
            pltpu.make_async_copy(rows_v.at[p], acc_sh.at[dst_v.at[p]], sems.at[p]).wait()

        plsc.subcore_barrier()

        start_idx(0, 0, dst_hbm, dst_v, semdst)
        start_idx(0, 0, ew_hbm, ew_v, semew)
        start_gather(0, 0)

        def chunk_body(j, p):
            nj = j + 1
            np_ = (p + 1) % RING

            @pl.when(nj < NCH)
            def _():
                @pl.when(nj >= RING)
                def _():
                    wait_scatter(nj - RING, np_)

                start_gather(nj, np_)
                start_idx(nj, np_, dst_hbm, dst_v, semdst)
                start_idx(nj, np_, ew_hbm, ew_v, semew)

            wait_gather(j, p)
            wait_idx(j, p, ew_hbm, ew_v, semew)

            def scale_group(g, carry):
                wv = ew_v[p, pl.ds(g * LANES, LANES)]
                for e in range(LANES):
                    w = jnp.broadcast_to(wv[e], (LANES,))
                    row = g * LANES + e
                    for d in range(D_BLKS):
                        rows_v[p, row, pl.ds(d * LANES, LANES)] = (
                            rows_v[p, row, pl.ds(d * LANES, LANES)] * w
                        )
                return carry

            lax.fori_loop(0, CHUNK // LANES, scale_group, 0)
            wait_idx(j, p, dst_hbm, dst_v, semdst)
            start_scatter(j, p)

        def ring_step(t, carry):
            for i in range(RING):
                chunk_body(t * RING + i, i)
            return carry

        full = (NCH // RING) * RING
        lax.fori_loop(0, NCH // RING, ring_step, 0)
        for j in range(full, NCH):
            chunk_body(jnp.int32(j), j % RING)

        for j in range(NCH - RING, NCH):
            wait_scatter(jnp.int32(j), j % RING)

        plsc.subcore_barrier()

        pltpu.sync_copy(
            acc_sh.at[pl.ds(row0, ROWS_PER_TILE)],
            out_hbm.at[c, pl.ds(row0, ROWS_PER_TILE)],
        )

    return k(x, src, dst, ew)


def _tc_finish(parts, W, b2):
    blk = 1000

    def body(p_ref, w_ref, b_ref, o_ref):
        acc = p_ref[0] + p_ref[1]
        h = jnp.dot(acc, w_ref[...], preferred_element_type=jnp.float32)
        o_ref[...] = jnp.maximum(h + b_ref[...], 0.0)

    return pl.pallas_call(
        body,
        grid=(N // blk,),
        in_specs=[
            pl.BlockSpec((NUM_CORES, blk, D), lambda i: (0, i, 0)),
            pl.BlockSpec((D, D), lambda i: (0, 0)),
            pl.BlockSpec((1, D), lambda i: (0, 0)),
        ],
        out_specs=pl.BlockSpec((blk, D), lambda i: (i, 0)),
        out_shape=jax.ShapeDtypeStruct((N, D), jnp.float32),
    )(parts, W, b2)


def kernel(x, edge_index, edge_weight, W, b):
    ei = edge_index.astype(jnp.int32)
    parts = _sc_aggregate(x, ei[0], ei[1], edge_weight)
    return _tc_finish(parts, W, b.reshape(1, D))
